# baseline (speedup 1.0000x reference)
import functools

import jax
import jax.numpy as jnp
from jax.experimental import pallas as pl
from jax.experimental.pallas import tpu as pltpu

_LEAKY_SLOPE = 0.2
_BN_EPS = 1e-5
_CB = 16
_NB_NODES = 4
_DEG = _NB_NODES + 1

_CELL_ARCH = (
    (0, 1, 0), (0, 2, 1), (1, 2, 2), (1, 3, 3), (2, 3, 4), (0, 4, 5),
    (3, 4, 6),
)


def _build_links():
    d = {}
    for src, dst, w in _CELL_ARCH:
        d.setdefault(dst, []).append((src, w))
    return tuple(tuple(d[k]) for k in range(1, _NB_NODES + 1))


_LINKS = _build_links()

_ROW_BV, _ROW_GV, _ROW_BETAV, _ROW_BS, _ROW_GE, _ROW_BETAE = range(6)
_ROW_WV = 8
_ROW_WS = 136
_SLAB_ROWS = 232


def _state_poly_coeffs(wt_ref):
    coeffs = [[1.0, 0.0, 0.0, 0.0, 0.0]]
    for dst_links in _LINKS:
        acc = [0.0] * _DEG
        for s, w in dst_links:
            w1 = wt_ref[w, 1]
            w2 = wt_ref[w, 2]
            c = coeffs[s]
            for j in range(_DEG):
                if isinstance(c[j], float) and c[j] == 0.0:
                    continue
                acc[j] = acc[j] + w1 * c[j]
                acc[j + 1] = acc[j + 1] + w2 * c[j]
        coeffs.append(acc)
    return coeffs


def _kernel_body(wt_ref, a_ref, g_ref, v_ref, e_ref, p_ref,
                 vout_ref, eout_ref, wk_ref, *, node_dim, edge_dim, slope,
                 eps):
    d, de = node_dim, edge_dim
    m = e_ref.shape[1]

    @pl.when(pl.program_id(0) == 0)
    def _fold():
        coeffs = _state_poly_coeffs(wt_ref)
        for j in range(_DEG):
            acc = None
            for k in range(_NB_NODES):
                c = coeffs[k + 1][j]
                if isinstance(c, float) and c == 0.0:
                    continue
                term = c * p_ref[_ROW_WV + k * d:_ROW_WV + (k + 1) * d, :d]
                acc = term if acc is None else acc + term
            wk_ref[j * d:(j + 1) * d, :] = acc

    bv = p_ref[_ROW_BV:_ROW_BV + 1, :d]
    gv = p_ref[_ROW_GV:_ROW_GV + 1, :d]
    betav = p_ref[_ROW_BETAV:_ROW_BETAV + 1, :d]
    bs = p_ref[_ROW_BS:_ROW_BS + 1, :de]
    ge = p_ref[_ROW_GE:_ROW_GE + 1, :de]
    betae = p_ref[_ROW_BETAE:_ROW_BETAE + 1, :de]

    wk = wk_ref[...]
    ws = p_ref[_ROW_WS:_ROW_WS + 2 * d + de, :de]

    def bn_leaky(h, g, b):
        inv_n = 1.0 / h.shape[0]
        s1 = jnp.sum(h, axis=0, keepdims=True)
        s2 = jnp.sum(h * h, axis=0, keepdims=True)
        mean = s1 * inv_n
        var = s2 * inv_n - mean * mean
        alpha = jax.lax.rsqrt(var + eps) * g
        beta = b - mean * alpha
        hn = h * alpha + beta
        return jnp.maximum(hn, slope * hn)

    kry = [[v_ref[c]] for c in range(_CB)]
    for _ in range(_NB_NODES):
        for c in range(_CB):
            kry[c].append(jnp.dot(a_ref[c], kry[c][-1],
                                  preferred_element_type=jnp.float32))

    v_lin = []
    for c in range(_CB):
        ck = jnp.concatenate(kry[c], axis=1)
        v_lin.append(jnp.dot(ck, wk,
                             preferred_element_type=jnp.float32) + bv)

    vg = [jnp.dot(g_ref[c], v_lin[c], preferred_element_type=jnp.float32)
          for c in range(_CB)]

    e_lin = []
    for c in range(_CB):
        e_in = e_ref[c]
        e_act = jnp.maximum(e_in, slope * e_in)
        cat = jnp.concatenate([vg[c][:m], e_act, vg[c][m:]], axis=1)
        e_lin.append(jnp.dot(cat, ws, preferred_element_type=jnp.float32)
                     + bs)

    for c in range(_CB):
        vout_ref[c] = bn_leaky(v_lin[c], gv, betav)
        eout_ref[c] = bn_leaky(e_lin[c], ge, betae)


def kernel(Wv, bv, Ws, bs, gv, betav, ge, betae, weight,
           a_mean_b, s_gather_b, v_in, e_in):
    b, n, d = v_in.shape
    _, m, de = e_in.shape
    rows = jnp.concatenate(
        [bv, gv, betav, bs, ge, betae,
         jnp.zeros((_ROW_WV - 6, d), jnp.float32), Wv, Ws], axis=0)
    p_slab = jax.lax.pad(rows, jnp.float32(0.0),
                         ((0, 0, 0), (0, 128 - d, 0)))

    body = functools.partial(_kernel_body, node_dim=d, edge_dim=de,
                             slope=_LEAKY_SLOPE, eps=_BN_EPS)

    smem = pltpu.MemorySpace.SMEM
    in_specs = [
        pl.BlockSpec(memory_space=smem),
        pl.BlockSpec((_CB, n, n), lambda i: (i, 0, 0)),
        pl.BlockSpec((_CB, 2 * m, n), lambda i: (i, 0, 0)),
        pl.BlockSpec((_CB, n, d), lambda i: (i, 0, 0)),
        pl.BlockSpec((_CB, m, de), lambda i: (i, 0, 0)),
        pl.BlockSpec(p_slab.shape, lambda i: (0, 0)),
    ]
    out_specs = (
        pl.BlockSpec((_CB, n, d), lambda i: (i, 0, 0)),
        pl.BlockSpec((_CB, m, de), lambda i: (i, 0, 0)),
    )
    out_shape = (jax.ShapeDtypeStruct((b, n, d), jnp.float32),
                 jax.ShapeDtypeStruct((b, m, de), jnp.float32))

    flops_per_cell = (2 * 4 * n * n * d + 2 * n * (_DEG * d) * d
                      + 2 * (2 * m) * n * d + 2 * m * (2 * d + de) * de
                      + 12 * (n * d + m * de))
    bytes_accessed = 4 * (a_mean_b.size + s_gather_b.size + v_in.size
                          + e_in.size + Wv.size + Ws.size
                          + b * n * d + b * m * de)

    vact, eact = pl.pallas_call(
        body,
        grid=(b // _CB,),
        in_specs=in_specs,
        out_specs=out_specs,
        out_shape=out_shape,
        scratch_shapes=[pltpu.VMEM((_DEG * d, d), jnp.float32)],
        compiler_params=pltpu.CompilerParams(
            dimension_semantics=("arbitrary",)),
        cost_estimate=pl.CostEstimate(
            flops=int(b * flops_per_cell),
            transcendentals=int(b * (d + de)),
            bytes_accessed=int(bytes_accessed)),
    )(weight, a_mean_b, s_gather_b, v_in, e_in, p_slab)
    return vact + v_in, eact + e_in


# speedup vs baseline: 1.3348x; 1.3348x over previous
import functools

import jax
import jax.numpy as jnp
from jax.experimental import pallas as pl
from jax.experimental.pallas import tpu as pltpu

_LEAKY_SLOPE = 0.2
_BN_EPS = 1e-5
_CB = 16
_NB_NODES = 4
_DEG = _NB_NODES + 1

_CELL_ARCH = (
    (0, 1, 0), (0, 2, 1), (1, 2, 2), (1, 3, 3), (2, 3, 4), (0, 4, 5),
    (3, 4, 6),
)


def _build_links():
    d = {}
    for src, dst, w in _CELL_ARCH:
        d.setdefault(dst, []).append((src, w))
    return tuple(tuple(d[k]) for k in range(1, _NB_NODES + 1))


_LINKS = _build_links()

_ROW_BV, _ROW_GV, _ROW_BETAV, _ROW_BS, _ROW_GE, _ROW_BETAE = range(6)
_ROW_WV = 8
_ROW_WS = 136
_SLAB_ROWS = 232


def _state_poly_coeffs(wt_ref):
    coeffs = [[1.0, 0.0, 0.0, 0.0, 0.0]]
    for dst_links in _LINKS:
        acc = [0.0] * _DEG
        for s, w in dst_links:
            w1 = wt_ref[w, 1]
            w2 = wt_ref[w, 2]
            c = coeffs[s]
            for j in range(_DEG):
                if isinstance(c[j], float) and c[j] == 0.0:
                    continue
                acc[j] = acc[j] + w1 * c[j]
                acc[j + 1] = acc[j + 1] + w2 * c[j]
        coeffs.append(acc)
    return coeffs


def _kernel_body(wt_ref, a_ref, g_ref, v_ref, e_ref, p_ref,
                 vout_ref, eout_ref, wk_ref, *, node_dim, edge_dim, slope,
                 eps):
    d, de = node_dim, edge_dim
    m = e_ref.shape[1]

    @pl.when(pl.program_id(0) == 0)
    def _fold():
        coeffs = _state_poly_coeffs(wt_ref)
        for j in range(_DEG):
            acc = None
            for k in range(_NB_NODES):
                c = coeffs[k + 1][j]
                if isinstance(c, float) and c == 0.0:
                    continue
                term = c * p_ref[_ROW_WV + k * d:_ROW_WV + (k + 1) * d, :d]
                acc = term if acc is None else acc + term
            wk_ref[j * d:(j + 1) * d, :] = acc

    bv = p_ref[_ROW_BV:_ROW_BV + 1, :d]
    gv = p_ref[_ROW_GV:_ROW_GV + 1, :d]
    betav = p_ref[_ROW_BETAV:_ROW_BETAV + 1, :d]
    bs = p_ref[_ROW_BS:_ROW_BS + 1, :de]
    ge = p_ref[_ROW_GE:_ROW_GE + 1, :de]
    betae = p_ref[_ROW_BETAE:_ROW_BETAE + 1, :de]

    wk = wk_ref[...]
    ws = p_ref[_ROW_WS:_ROW_WS + 2 * d + de, :de]

    def bn_leaky_res(h, g, b, res):
        inv_n = 1.0 / h.shape[0]
        s1 = jnp.sum(h, axis=0, keepdims=True)
        s2 = jnp.sum(h * h, axis=0, keepdims=True)
        mean = s1 * inv_n
        var = s2 * inv_n - mean * mean
        alpha = jax.lax.rsqrt(var + eps) * g
        beta = b - mean * alpha
        hn = h * alpha + beta
        return jnp.maximum(hn, slope * hn) + res

    kry = [[v_ref[c]] for c in range(_CB)]
    for _ in range(_NB_NODES):
        for c in range(_CB):
            kry[c].append(jnp.dot(a_ref[c], kry[c][-1],
                                  preferred_element_type=jnp.float32))

    v_lin = []
    for c in range(_CB):
        ck = jnp.concatenate(kry[c], axis=1)
        v_lin.append(jnp.dot(ck, wk,
                             preferred_element_type=jnp.float32) + bv)

    vg = [jnp.dot(g_ref[c], v_lin[c], preferred_element_type=jnp.float32)
          for c in range(_CB)]

    e_lin = []
    for c in range(_CB):
        e_in = e_ref[c]
        e_act = jnp.maximum(e_in, slope * e_in)
        cat = jnp.concatenate([vg[c][:m], e_act, vg[c][m:]], axis=1)
        e_lin.append(jnp.dot(cat, ws, preferred_element_type=jnp.float32)
                     + bs)

    for c in range(_CB):
        vout_ref[c] = bn_leaky_res(v_lin[c], gv, betav, v_ref[c])
        eout_ref[c] = bn_leaky_res(e_lin[c], ge, betae, e_ref[c])


def kernel(Wv, bv, Ws, bs, gv, betav, ge, betae, weight,
           a_mean_b, s_gather_b, v_in, e_in):
    b, n, d = v_in.shape
    _, m, de = e_in.shape
    rows = jnp.concatenate(
        [bv, gv, betav, bs, ge, betae,
         jnp.zeros((_ROW_WV - 6, d), jnp.float32), Wv, Ws], axis=0)
    p_slab = jax.lax.pad(rows, jnp.float32(0.0),
                         ((0, 0, 0), (0, 128 - d, 0)))

    body = functools.partial(_kernel_body, node_dim=d, edge_dim=de,
                             slope=_LEAKY_SLOPE, eps=_BN_EPS)

    smem = pltpu.MemorySpace.SMEM
    in_specs = [
        pl.BlockSpec(memory_space=smem),
        pl.BlockSpec((_CB, n, n), lambda i: (i, 0, 0)),
        pl.BlockSpec((_CB, 2 * m, n), lambda i: (i, 0, 0)),
        pl.BlockSpec((_CB, n, d), lambda i: (i, 0, 0)),
        pl.BlockSpec((_CB, m, de), lambda i: (i, 0, 0)),
        pl.BlockSpec(p_slab.shape, lambda i: (0, 0)),
    ]
    out_specs = (
        pl.BlockSpec((_CB, n, d), lambda i: (i, 0, 0)),
        pl.BlockSpec((_CB, m, de), lambda i: (i, 0, 0)),
    )
    out_shape = (jax.ShapeDtypeStruct((b, n, d), jnp.float32),
                 jax.ShapeDtypeStruct((b, m, de), jnp.float32))

    flops_per_cell = (2 * 4 * n * n * d + 2 * n * (_DEG * d) * d
                      + 2 * (2 * m) * n * d + 2 * m * (2 * d + de) * de
                      + 12 * (n * d + m * de))
    bytes_accessed = 4 * (a_mean_b.size + s_gather_b.size + v_in.size
                          + e_in.size + Wv.size + Ws.size
                          + b * n * d + b * m * de)

    return pl.pallas_call(
        body,
        grid=(b // _CB,),
        in_specs=in_specs,
        out_specs=out_specs,
        out_shape=out_shape,
        scratch_shapes=[pltpu.VMEM((_DEG * d, d), jnp.float32)],
        compiler_params=pltpu.CompilerParams(
            dimension_semantics=("arbitrary",)),
        cost_estimate=pl.CostEstimate(
            flops=int(b * flops_per_cell),
            transcendentals=int(b * (d + de)),
            bytes_accessed=int(bytes_accessed)),
    )(weight, a_mean_b, s_gather_b, v_in, e_in, p_slab)
